# baseline (device time: 45754 ns/iter reference)
import jax
import jax.numpy as jnp
from jax import lax
from jax.experimental import pallas as pl
from jax.experimental.pallas import tpu as pltpu

N_DEV = 4
N_CHUNKS = 4


def kernel(x, w_mat, scale_x, scale_w):
    m_total, k_loc = x.shape
    _, n = w_mat.shape
    m_per = m_total // N_DEV
    m_ch = m_per // N_CHUNKS

    def body(x_hbm, w_hbm, sx_ref, sw_ref, out_ref,
             x_diag, w_vmem, comm_ref, diag_ref,
             xw_sems, send_sems, recv_sems, dsend_sems, drecv_sems):
        my = lax.axis_index("i")

        x_cp = pltpu.make_async_copy(
            x_hbm.at[pl.ds(my * m_per, m_per), :], x_diag, xw_sems.at[0])
        x_cp.start()
        w_copies = []
        for s, d in enumerate((0, 1, 3, 2)):
            src = (my - d) % N_DEV
            cp = pltpu.make_async_copy(
                w_hbm.at[pl.ds(src * k_loc, k_loc), :],
                w_vmem.at[s], xw_sems.at[1 + s])
            cp.start()
            w_copies.append(cp)

        barrier_sem = pltpu.get_barrier_semaphore()
        for d in range(1, N_DEV):
            pl.semaphore_signal(
                barrier_sem, inc=1,
                device_id=((my + d) % N_DEV,),
                device_id_type=pl.DeviceIdType.MESH,
            )
        pl.semaphore_wait(barrier_sem, N_DEV - 1)

        phase_a = []
        for s, d in enumerate((1, 3)):
            tgt = (my + d) % N_DEV
            rdma = pltpu.make_async_remote_copy(
                src_ref=x_hbm.at[pl.ds(tgt * m_per, m_per), :],
                dst_ref=comm_ref.at[s],
                send_sem=send_sems.at[s],
                recv_sem=recv_sems.at[s],
                device_id=(tgt,),
                device_id_type=pl.DeviceIdType.MESH,
            )
            rdma.start()
            phase_a.append(rdma)

        x_cp.wait()
        w_copies[0].wait()
        acc = jnp.dot(x_diag[:, :], w_vmem[0],
                      preferred_element_type=jnp.int32)

        diag_tgt = (my + 2) % N_DEV
        phase_b = []
        for r in phase_a:
            r.wait_send()
        for c in range(N_CHUNKS):
            rdma = pltpu.make_async_remote_copy(
                src_ref=x_hbm.at[
                    pl.ds(diag_tgt * m_per + c * m_ch, m_ch), :],
                dst_ref=diag_ref.at[c],
                send_sem=dsend_sems.at[c],
                recv_sem=drecv_sems.at[c],
                device_id=(diag_tgt,),
                device_id_type=pl.DeviceIdType.MESH,
            )
            rdma.start()
            phase_b.append(rdma)

        for s, d in enumerate((1, 3)):
            phase_a[s].wait_recv()
            w_copies[1 + s].wait()
            acc = acc + jnp.dot(comm_ref[s], w_vmem[1 + s],
                                preferred_element_type=jnp.int32)

        w_copies[3].wait()
        scale = sx_ref[0] * sw_ref[0]
        for c in range(N_CHUNKS):
            phase_b[c].wait_recv()
            part = jnp.dot(diag_ref[c], w_vmem[3],
                           preferred_element_type=jnp.int32)
            rows = acc[c * m_ch:(c + 1) * m_ch, :] + part
            out_ref[pl.ds(c * m_ch, m_ch), :] = (
                rows.astype(jnp.float32) * scale)

        for c in range(N_CHUNKS):
            phase_b[c].wait_send()

    return pl.pallas_call(
        body,
        out_shape=jax.ShapeDtypeStruct((m_per, n), jnp.float32),
        in_specs=[
            pl.BlockSpec(memory_space=pl.ANY),
            pl.BlockSpec(memory_space=pl.ANY),
            pl.BlockSpec(memory_space=pltpu.SMEM),
            pl.BlockSpec(memory_space=pltpu.SMEM),
        ],
        out_specs=pl.BlockSpec(memory_space=pltpu.VMEM),
        scratch_shapes=[
            pltpu.VMEM((m_per, k_loc), jnp.int8),
            pltpu.VMEM((N_DEV, k_loc, n), jnp.int8),
            pltpu.VMEM((2, m_per, k_loc), jnp.int8),
            pltpu.VMEM((N_CHUNKS, m_ch, k_loc), jnp.int8),
            pltpu.SemaphoreType.DMA((1 + N_DEV,)),
            pltpu.SemaphoreType.DMA((2,)),
            pltpu.SemaphoreType.DMA((2,)),
            pltpu.SemaphoreType.DMA((N_CHUNKS,)),
            pltpu.SemaphoreType.DMA((N_CHUNKS,)),
        ],
        compiler_params=pltpu.CompilerParams(collective_id=0),
    )(x, w_mat, scale_x, scale_w)


# device time: 38645 ns/iter; 1.1840x vs baseline; 1.1840x over previous
import jax
import jax.numpy as jnp
from jax import lax
from jax.experimental import pallas as pl
from jax.experimental.pallas import tpu as pltpu

N_DEV = 4
N_CH = 4
M_CH = 256
DIAG_SIZES = (512, 256, 128, 128)
DIAG_OFFS = (0, 512, 768, 896)


def kernel(x, w_mat, scale_x, scale_w):
    m_total, k_loc = x.shape
    _, n = w_mat.shape
    m_per = m_total // N_DEV

    def body(x_ref, w_ref, sx_ref, sw_ref, out_ref,
             comm_ref, diag_ref, sendA, recvA, sendB, recvB):
        my = lax.axis_index("i")

        barrier_sem = pltpu.get_barrier_semaphore()
        for d in range(1, N_DEV):
            pl.semaphore_signal(
                barrier_sem, inc=1,
                device_id=((my + d) % N_DEV,),
                device_id_type=pl.DeviceIdType.MESH,
            )
        pl.semaphore_wait(barrier_sem, N_DEV - 1)

        phase_a = []
        for s, d in enumerate((1, 3)):
            tgt = (my + d) % N_DEV
            for c in range(N_CH):
                rdma = pltpu.make_async_remote_copy(
                    src_ref=x_ref.at[
                        pl.ds(tgt * m_per + c * M_CH, M_CH), :],
                    dst_ref=comm_ref.at[s, c],
                    send_sem=sendA.at[s * N_CH + c],
                    recv_sem=recvA.at[s * N_CH + c],
                    device_id=(tgt,),
                    device_id_type=pl.DeviceIdType.MESH,
                )
                rdma.start()
                phase_a.append(rdma)

        w_own = w_ref[pl.ds(my * k_loc, k_loc), :]
        acc = [None] * N_CH
        for c in range(N_CH):
            acc[c] = jnp.dot(
                x_ref[pl.ds(my * m_per + c * M_CH, M_CH), :], w_own,
                preferred_element_type=jnp.int32)

        w_nbr = [w_ref[pl.ds(((my - 1) % N_DEV) * k_loc, k_loc), :],
                 w_ref[pl.ds(((my + 1) % N_DEV) * k_loc, k_loc), :]]
        for c in range(2):
            for s in range(2):
                phase_a[s * N_CH + c].wait_recv()
                acc[c] = acc[c] + jnp.dot(
                    comm_ref[s, c], w_nbr[s],
                    preferred_element_type=jnp.int32)

        for r in phase_a:
            r.wait_send()
        diag_tgt = (my + 2) % N_DEV
        phase_b = []
        for c in range(len(DIAG_SIZES)):
            rdma = pltpu.make_async_remote_copy(
                src_ref=x_ref.at[
                    pl.ds(diag_tgt * m_per + DIAG_OFFS[c], DIAG_SIZES[c]), :],
                dst_ref=diag_ref.at[pl.ds(DIAG_OFFS[c], DIAG_SIZES[c]), :],
                send_sem=sendB.at[c],
                recv_sem=recvB.at[c],
                device_id=(diag_tgt,),
                device_id_type=pl.DeviceIdType.MESH,
            )
            rdma.start()
            phase_b.append(rdma)

        for c in range(2, N_CH):
            for s in range(2):
                phase_a[s * N_CH + c].wait_recv()
                acc[c] = acc[c] + jnp.dot(
                    comm_ref[s, c], w_nbr[s],
                    preferred_element_type=jnp.int32)

        w_diag = w_ref[pl.ds(diag_tgt * k_loc, k_loc), :]
        scale = sx_ref[0] * sw_ref[0]
        for c in range(len(DIAG_SIZES)):
            phase_b[c].wait_recv()
            off, sz = DIAG_OFFS[c], DIAG_SIZES[c]
            part = jnp.dot(diag_ref[pl.ds(off, sz), :], w_diag,
                           preferred_element_type=jnp.int32)
            a0 = off // M_CH
            if sz > M_CH:
                base = jnp.concatenate(acc[a0:a0 + sz // M_CH], axis=0)
            elif sz == M_CH:
                base = acc[a0]
            else:
                r0 = off % M_CH
                base = acc[a0][r0:r0 + sz, :]
            out_ref[pl.ds(off, sz), :] = (
                (base + part).astype(jnp.float32) * scale)

        for c in range(len(DIAG_SIZES)):
            phase_b[c].wait_send()

    return pl.pallas_call(
        body,
        out_shape=jax.ShapeDtypeStruct((m_per, n), jnp.float32),
        in_specs=[
            pl.BlockSpec(memory_space=pltpu.VMEM),
            pl.BlockSpec(memory_space=pltpu.VMEM),
            pl.BlockSpec(memory_space=pltpu.SMEM),
            pl.BlockSpec(memory_space=pltpu.SMEM),
        ],
        out_specs=pl.BlockSpec(memory_space=pltpu.VMEM),
        scratch_shapes=[
            pltpu.VMEM((2, N_CH, M_CH, k_loc), jnp.int8),
            pltpu.VMEM((m_per, k_loc), jnp.int8),
            pltpu.SemaphoreType.DMA((2 * N_CH,)),
            pltpu.SemaphoreType.DMA((2 * N_CH,)),
            pltpu.SemaphoreType.DMA((len(DIAG_SIZES),)),
            pltpu.SemaphoreType.DMA((len(DIAG_SIZES),)),
        ],
        compiler_params=pltpu.CompilerParams(collective_id=0),
    )(x, w_mat, scale_x, scale_w)
